# baseline (speedup 1.0000x reference)
import functools

import jax
import jax.numpy as jnp
from jax import lax
from jax.experimental import pallas as pl
from jax.experimental.pallas import tpu as pltpu
from jax.experimental.pallas import tpu_sc as plsc


_CHUNK = 128
_GROUP = 5
_NBUF = 2
_CB = 2048


def _densify_block(x_ref, o_ref):
    eye = jnp.eye(64, dtype=jnp.float32)
    xt = lax.dot_general(
        x_ref[...], eye, (((0,), (0,)), ((), ())),
        precision=lax.Precision.HIGHEST,
        preferred_element_type=jnp.float32,
    )
    xt3 = xt.reshape(_CB // 2, 2, 64)
    o_ref[...] = jnp.concatenate([xt3[:, 0, :], xt3[:, 1, :]], axis=1)


def _densify(table_t):
    d, v = table_t.shape
    grid = (v + _CB - 1) // _CB
    return pl.pallas_call(
        _densify_block,
        grid=(grid,),
        in_specs=[pl.BlockSpec((d, _CB), lambda i: (0, i))],
        out_specs=pl.BlockSpec((_CB // 2, 128), lambda i: (i, 0)),
        out_shape=jax.ShapeDtypeStruct((v // 2, 128), jnp.float32),
    )(table_t)


@functools.partial(jax.jit, static_argnames=("n_workers", "n_groups", "d"))
def _gather_sc(idx_flat, table_t, n_workers, n_groups, d):
    dense = _densify(table_t).reshape(table_t.shape[1], d)

    mesh = plsc.VectorSubcoreMesh(core_axis_name="c", subcore_axis_name="s")
    nc = mesh.num_cores
    rows_per_group = _GROUP * _CHUNK
    b_per_w = n_groups * rows_per_group

    @functools.partial(
        pl.kernel,
        out_type=jax.ShapeDtypeStruct((n_workers * b_per_w, d), jnp.float32),
        mesh=mesh,
        scratch_types=[
            pltpu.VMEM((n_groups * _GROUP * _CHUNK,), jnp.int32),
            pltpu.VMEM((_NBUF, rows_per_group, d), jnp.float32),
            pltpu.SemaphoreType.DMA,
            pltpu.SemaphoreType.DMA,
            pltpu.SemaphoreType.DMA,
        ],
        compiler_params=pltpu.CompilerParams(use_tc_tiling_on_sc=False),
    )
    def k(idx_hbm, table_hbm, out_hbm, idx_v, rows_v, gsem, osem0, osem1):
        wid = lax.axis_index("s") * nc + lax.axis_index("c")
        base = wid * b_per_w
        pltpu.sync_copy(idx_hbm.at[pl.ds(base, b_per_w)], idx_v)
        osems = (osem0, osem1)

        def group(g, _):
            def for_buf(buf):
                @pl.when(g >= _NBUF)
                def _():
                    pltpu.make_async_copy(
                        rows_v.at[buf],
                        out_hbm.at[pl.ds(base + (g - _NBUF) * rows_per_group,
                                         rows_per_group)],
                        osems[buf],
                    ).wait()

                for c in range(_GROUP):
                    pltpu.make_async_copy(
                        table_hbm.at[idx_v.at[pl.ds((g * _GROUP + c) * _CHUNK, _CHUNK)]],
                        rows_v.at[buf, pl.ds(c * _CHUNK, _CHUNK)],
                        gsem,
                    ).start()
                for c in range(_GROUP):
                    pltpu.make_async_copy(
                        table_hbm.at[idx_v.at[pl.ds((g * _GROUP + c) * _CHUNK, _CHUNK)]],
                        rows_v.at[buf, pl.ds(c * _CHUNK, _CHUNK)],
                        gsem,
                    ).wait()

                pltpu.make_async_copy(
                    rows_v.at[buf],
                    out_hbm.at[pl.ds(base + g * rows_per_group, rows_per_group)],
                    osems[buf],
                ).start()

            for buf in range(_NBUF):
                pl.when(lax.rem(g, _NBUF) == buf)(lambda b=buf: for_buf(b))
            return 0

        lax.fori_loop(0, n_groups, group, 0)

        for t in range(_NBUF):
            g = n_groups - _NBUF + t
            pltpu.make_async_copy(
                rows_v.at[g % _NBUF],
                out_hbm.at[pl.ds(base + g * rows_per_group, rows_per_group)],
                osems[g % _NBUF],
            ).wait()

    return k(idx_flat, dense)


def kernel(indices, table):
    b, l = indices.shape
    v, d = table.shape
    total = b * l
    info = plsc.get_sparse_core_info()
    n_workers = info.num_cores * info.num_subcores
    rows_per_group = _GROUP * _CHUNK
    assert total % (n_workers * rows_per_group) == 0
    n_groups = total // (n_workers * rows_per_group)
    idx_flat = indices.reshape(total).astype(jnp.int32)
    out = _gather_sc(idx_flat, table.T, n_workers, n_groups, d)
    return out.reshape(b, l, d)


# speedup vs baseline: 1.5321x; 1.5321x over previous
import functools

import jax
import jax.numpy as jnp
from jax import lax
from jax.experimental import pallas as pl
from jax.experimental.pallas import tpu as pltpu
from jax.experimental.pallas import tpu_sc as plsc


_CHUNK = 128
_GROUP = 5
_NBUF = 2
_CB = 2048


def _densify_block(x_ref, o_ref):
    xt = x_ref[...].T
    o_ref[:, 0:64] = xt[: _CB // 2]
    o_ref[:, 64:128] = xt[_CB // 2 :]


def _densify(table_t):
    d, v = table_t.shape
    grid = (v + _CB - 1) // _CB
    return pl.pallas_call(
        _densify_block,
        grid=(grid,),
        in_specs=[pl.BlockSpec((d, _CB), lambda i: (0, i))],
        out_specs=pl.BlockSpec((_CB // 2, 128), lambda i: (i, 0)),
        out_shape=jax.ShapeDtypeStruct((grid * _CB // 2, 128), jnp.float32),
    )(table_t)


@functools.partial(jax.jit, static_argnames=("n_workers", "n_groups", "d"))
def _gather_sc(idx_flat, table_t, n_workers, n_groups, d):
    dense = _densify(table_t)
    dense = dense.reshape(dense.shape[0] * 2, d)

    mesh = plsc.VectorSubcoreMesh(core_axis_name="c", subcore_axis_name="s")
    nc = mesh.num_cores
    rows_per_group = _GROUP * _CHUNK
    b_per_w = n_groups * rows_per_group

    @functools.partial(
        pl.kernel,
        out_type=jax.ShapeDtypeStruct((n_workers * b_per_w, d), jnp.float32),
        mesh=mesh,
        scratch_types=[
            pltpu.VMEM((n_groups * _GROUP * _CHUNK,), jnp.int32),
            pltpu.VMEM((_NBUF, rows_per_group, d), jnp.float32),
            pltpu.SemaphoreType.DMA,
            pltpu.SemaphoreType.DMA,
            pltpu.SemaphoreType.DMA,
        ],
        compiler_params=pltpu.CompilerParams(use_tc_tiling_on_sc=False),
    )
    def k(idx_hbm, table_hbm, out_hbm, idx_v, rows_v, gsem, osem0, osem1):
        wid = lax.axis_index("s") * nc + lax.axis_index("c")
        base = wid * b_per_w
        pltpu.sync_copy(idx_hbm.at[pl.ds(base, b_per_w)], idx_v)
        osems = (osem0, osem1)

        def group(g, _):
            def for_buf(buf):
                @pl.when(g >= _NBUF)
                def _():
                    pltpu.make_async_copy(
                        rows_v.at[buf],
                        out_hbm.at[pl.ds(base + (g - _NBUF) * rows_per_group,
                                         rows_per_group)],
                        osems[buf],
                    ).wait()

                for c in range(_GROUP):
                    pltpu.make_async_copy(
                        table_hbm.at[idx_v.at[pl.ds((g * _GROUP + c) * _CHUNK, _CHUNK)]],
                        rows_v.at[buf, pl.ds(c * _CHUNK, _CHUNK)],
                        gsem,
                    ).start()
                for c in range(_GROUP):
                    pltpu.make_async_copy(
                        table_hbm.at[idx_v.at[pl.ds((g * _GROUP + c) * _CHUNK, _CHUNK)]],
                        rows_v.at[buf, pl.ds(c * _CHUNK, _CHUNK)],
                        gsem,
                    ).wait()

                pltpu.make_async_copy(
                    rows_v.at[buf],
                    out_hbm.at[pl.ds(base + g * rows_per_group, rows_per_group)],
                    osems[buf],
                ).start()

            for buf in range(_NBUF):
                pl.when(lax.rem(g, _NBUF) == buf)(lambda b=buf: for_buf(b))
            return 0

        lax.fori_loop(0, n_groups, group, 0)

        for t in range(_NBUF):
            g = n_groups - _NBUF + t
            pltpu.make_async_copy(
                rows_v.at[g % _NBUF],
                out_hbm.at[pl.ds(base + g * rows_per_group, rows_per_group)],
                osems[g % _NBUF],
            ).wait()

    return k(idx_flat, dense)


def kernel(indices, table):
    b, l = indices.shape
    v, d = table.shape
    total = b * l
    info = plsc.get_sparse_core_info()
    n_workers = info.num_cores * info.num_subcores
    rows_per_group = _GROUP * _CHUNK
    assert total % (n_workers * rows_per_group) == 0
    n_groups = total // (n_workers * rows_per_group)
    idx_flat = indices.reshape(total).astype(jnp.int32)
    j = idx_flat & (_CB - 1)
    idx_flat = idx_flat + j - jnp.where(j < _CB // 2, 0, _CB - 1)
    out = _gather_sc(idx_flat, table.T, n_workers, n_groups, d)
    return out.reshape(b, l, d)
